# Initial kernel scaffold; baseline (speedup 1.0000x reference)
import jax
import jax.numpy as jnp
from jax.experimental import pallas as pl


def kernel(x0, x1, x2, W_self0, W_agg0, W_self1, W_agg1):
    raise NotImplementedError("write your pallas kernel here")


# speedup vs baseline: 28.8229x; 28.8229x over previous
import functools

import jax
import jax.numpy as jnp
from jax.experimental import pallas as pl

_D = 128
_K = 16


def _gelu_exact(x):
    return 0.5 * x * (1.0 + jax.lax.erf(x * 0.7071067811865476))


def _fused_body(x0_ref, x1_ref, ws0_ref, ws1_ref, wa1_ref, out_ref):
    ws0 = ws0_ref[...]
    b = _gelu_exact(jnp.dot(x1_ref[...], ws0, preferred_element_type=jnp.float32))
    n = x0_ref.shape[0]
    s = b.reshape(n, _K, _D).sum(axis=1)
    a = _gelu_exact(jnp.dot(x0_ref[...], ws0, preferred_element_type=jnp.float32))
    out_ref[:, :_D] = jnp.dot(a, ws1_ref[...], preferred_element_type=jnp.float32)
    out_ref[:, _D:] = jnp.dot(s, wa1_ref[...], preferred_element_type=jnp.float32)


@functools.partial(jax.jit, static_argnames=("blk",))
def _run(x0, x1, w_self0, w_self1, w_agg1, blk=256):
    n0 = x0.shape[0]
    grid = (n0 // blk,)
    return pl.pallas_call(
        _fused_body,
        grid=grid,
        in_specs=[
            pl.BlockSpec((blk, _D), lambda i: (i, 0)),
            pl.BlockSpec((blk * _K, _D), lambda i: (i, 0)),
            pl.BlockSpec((_D, _D), lambda i: (0, 0)),
            pl.BlockSpec((_D, _D), lambda i: (0, 0)),
            pl.BlockSpec((_D, _D), lambda i: (0, 0)),
        ],
        out_specs=pl.BlockSpec((blk, 2 * _D), lambda i: (i, 0)),
        out_shape=jax.ShapeDtypeStruct((n0, 2 * _D), jnp.float32),
    )(x0, x1, w_self0, w_self1, w_agg1)


def kernel(x0, x1, x2, W_self0, W_agg0, W_self1, W_agg1):
    del x2, W_agg0
    return _run(x0, x1, W_self0, W_self1, W_agg1)


# speedup vs baseline: 34.0225x; 1.1804x over previous
import functools

import jax
import jax.numpy as jnp
from jax.experimental import pallas as pl

_D = 128
_K = 16


def _gelu_exact(x):
    return 0.5 * x * (1.0 + jax.lax.erf(x * 0.7071067811865476))


def _fused_body(x0_ref, x1_ref, ws0_ref, ws1_ref, wa1_ref, out_ref):
    ws0 = ws0_ref[...]
    b = _gelu_exact(jnp.dot(x1_ref[...], ws0, preferred_element_type=jnp.float32))
    n = x0_ref.shape[0]
    s = b.reshape(n, _K, _D).sum(axis=1)
    a = _gelu_exact(jnp.dot(x0_ref[...], ws0, preferred_element_type=jnp.float32))
    out_ref[:, :_D] = jnp.dot(a, ws1_ref[...], preferred_element_type=jnp.float32)
    out_ref[:, _D:] = jnp.dot(s, wa1_ref[...], preferred_element_type=jnp.float32)


@functools.partial(jax.jit, static_argnames=("blk",))
def _run(x0, x1, w_self0, w_self1, w_agg1, blk=512):
    n0 = x0.shape[0]
    grid = (n0 // blk,)
    return pl.pallas_call(
        _fused_body,
        grid=grid,
        in_specs=[
            pl.BlockSpec((blk, _D), lambda i: (i, 0)),
            pl.BlockSpec((blk * _K, _D), lambda i: (i, 0)),
            pl.BlockSpec((_D, _D), lambda i: (0, 0)),
            pl.BlockSpec((_D, _D), lambda i: (0, 0)),
            pl.BlockSpec((_D, _D), lambda i: (0, 0)),
        ],
        out_specs=pl.BlockSpec((blk, 2 * _D), lambda i: (i, 0)),
        out_shape=jax.ShapeDtypeStruct((n0, 2 * _D), jnp.float32),
    )(x0, x1, w_self0, w_self1, w_agg1)


def kernel(x0, x1, x2, W_self0, W_agg0, W_self1, W_agg1):
    del x2, W_agg0
    return _run(x0, x1, W_self0, W_self1, W_agg1)


# speedup vs baseline: 34.3410x; 1.0094x over previous
import functools

import jax
import jax.numpy as jnp
from jax.experimental import pallas as pl

_D = 128
_K = 16


def _gelu_exact(x):
    return 0.5 * x * (1.0 + jax.lax.erf(x * 0.7071067811865476))


def _fused_body(x0_ref, x1_ref, ws0_ref, ws1_ref, wa1_ref, out_ref):
    ws0 = ws0_ref[...]
    b = _gelu_exact(jnp.dot(x1_ref[...], ws0, preferred_element_type=jnp.float32))
    n = x0_ref.shape[0]
    s = b.reshape(n, _K, _D).sum(axis=1)
    a = _gelu_exact(jnp.dot(x0_ref[...], ws0, preferred_element_type=jnp.float32))
    out_ref[:, :_D] = jnp.dot(a, ws1_ref[...], preferred_element_type=jnp.float32)
    out_ref[:, _D:] = jnp.dot(s, wa1_ref[...], preferred_element_type=jnp.float32)


@functools.partial(jax.jit, static_argnames=("blk",))
def _run(x0, x1, w_self0, w_self1, w_agg1, blk=1024):
    n0 = x0.shape[0]
    grid = (n0 // blk,)
    return pl.pallas_call(
        _fused_body,
        grid=grid,
        in_specs=[
            pl.BlockSpec((blk, _D), lambda i: (i, 0)),
            pl.BlockSpec((blk * _K, _D), lambda i: (i, 0)),
            pl.BlockSpec((_D, _D), lambda i: (0, 0)),
            pl.BlockSpec((_D, _D), lambda i: (0, 0)),
            pl.BlockSpec((_D, _D), lambda i: (0, 0)),
        ],
        out_specs=pl.BlockSpec((blk, 2 * _D), lambda i: (i, 0)),
        out_shape=jax.ShapeDtypeStruct((n0, 2 * _D), jnp.float32),
    )(x0, x1, w_self0, w_self1, w_agg1)


def kernel(x0, x1, x2, W_self0, W_agg0, W_self1, W_agg1):
    del x2, W_agg0
    return _run(x0, x1, W_self0, W_self1, W_agg1)
